# Initial kernel scaffold; baseline (speedup 1.0000x reference)
import jax
import jax.numpy as jnp
from jax.experimental import pallas as pl


def kernel(indices, table):
    raise NotImplementedError("write your pallas kernel here")


# speedup vs baseline: 3.1840x; 3.1840x over previous
import functools

import jax
import jax.numpy as jnp
from jax import lax
from jax.experimental import pallas as pl
from jax.experimental.pallas import tpu as pltpu
from jax.experimental.pallas import tpu_sc as plsc

D_MODEL = 64
MAX_LEN = 200
BATCH = 4096
NUM_WORKERS = 32
BB = BATCH // NUM_WORKERS
LANES = 16


def _pos_encoding():
    even_i = jnp.arange(0, D_MODEL, 2).astype(jnp.float32)
    denominator = jnp.power(10000.0, even_i / D_MODEL)
    position = jnp.arange(MAX_LEN, dtype=jnp.float32).reshape(MAX_LEN, 1)
    even_pe = jnp.sin(position / denominator)
    odd_pe = jnp.cos(position / denominator)
    return jnp.stack([even_pe, odd_pe], axis=2).reshape(MAX_LEN, D_MODEL)


def kernel(indices, table):
    pe = _pos_encoding()
    idx_t = indices.T

    mesh = plsc.VectorSubcoreMesh(core_axis_name="c", subcore_axis_name="s")

    @functools.partial(
        pl.kernel,
        mesh=mesh,
        compiler_params=pltpu.CompilerParams(use_tc_tiling_on_sc=False),
        out_type=jax.ShapeDtypeStruct((BATCH, MAX_LEN, D_MODEL), jnp.float32),
        scratch_types=[
            pltpu.VMEM((MAX_LEN, BB), jnp.int32),
            pltpu.VMEM((MAX_LEN, D_MODEL), jnp.float32),
            pltpu.VMEM((BB, D_MODEL), jnp.float32),
            pltpu.SemaphoreType.DMA,
        ],
    )
    def k(idx_hbm, table_hbm, pe_hbm, out_hbm, idx_v, pe_v, buf, sem):
        wid = lax.axis_index("s") * 2 + lax.axis_index("c")
        b0 = wid * BB
        pltpu.sync_copy(idx_hbm.at[:, pl.ds(b0, BB)], idx_v)
        pltpu.sync_copy(pe_hbm, pe_v)

        def step(t, carry):
            pltpu.async_copy(table_hbm.at[idx_v.at[t]], buf, sem).wait()
            pes = [pe_v[t, pl.ds(LANES * j, LANES)] for j in range(D_MODEL // LANES)]

            def row(r, c2):
                for j in range(D_MODEL // LANES):
                    plsc.addupdate(buf.at[r, pl.ds(LANES * j, LANES)], pes[j])
                return c2

            lax.fori_loop(0, BB, row, 0)
            pltpu.sync_copy(buf, out_hbm.at[pl.ds(b0, BB), t])
            return carry

        lax.fori_loop(0, MAX_LEN, step, 0)

    return k(idx_t, table, pe)


# speedup vs baseline: 4.1494x; 1.3032x over previous
import functools

import jax
import jax.numpy as jnp
from jax import lax
from jax.experimental import pallas as pl
from jax.experimental.pallas import tpu as pltpu
from jax.experimental.pallas import tpu_sc as plsc

D_MODEL = 64
MAX_LEN = 200
BATCH = 4096
NUM_WORKERS = 32
BPW = BATCH // NUM_WORKERS
LANES = 16
NBUF = 4
G1 = 128
G2 = MAX_LEN - G1


def _pos_encoding():
    even_i = jnp.arange(0, D_MODEL, 2).astype(jnp.float32)
    denominator = jnp.power(10000.0, even_i / D_MODEL)
    position = jnp.arange(MAX_LEN, dtype=jnp.float32).reshape(MAX_LEN, 1)
    even_pe = jnp.sin(position / denominator)
    odd_pe = jnp.cos(position / denominator)
    return jnp.stack([even_pe, odd_pe], axis=2).reshape(MAX_LEN, D_MODEL)


def kernel(indices, table):
    pe = _pos_encoding()

    mesh = plsc.VectorSubcoreMesh(core_axis_name="c", subcore_axis_name="s")

    @functools.partial(
        pl.kernel,
        mesh=mesh,
        compiler_params=pltpu.CompilerParams(use_tc_tiling_on_sc=False),
        out_type=jax.ShapeDtypeStruct((BATCH, MAX_LEN, D_MODEL), jnp.float32),
        scratch_types=[
            pltpu.VMEM((BPW, MAX_LEN), jnp.int32),
            pltpu.VMEM((MAX_LEN, D_MODEL), jnp.float32),
            pltpu.VMEM((NBUF, MAX_LEN, D_MODEL), jnp.float32),
            pltpu.SemaphoreType.DMA((NBUF,)),
            pltpu.SemaphoreType.DMA((NBUF,)),
        ],
    )
    def k(idx_hbm, table_hbm, pe_hbm, out_hbm, idx_v, pe_v, bufs, gsem, osem):
        wid = lax.axis_index("s") * 2 + lax.axis_index("c")
        b0 = wid * BPW
        pltpu.sync_copy(idx_hbm.at[pl.ds(b0, BPW)], idx_v)
        pltpu.sync_copy(pe_hbm, pe_v)

        def start_gathers(i, s):
            pltpu.async_copy(
                table_hbm.at[idx_v.at[i, pl.ds(0, G1)]],
                bufs.at[s, pl.ds(0, G1)], gsem.at[s])
            pltpu.async_copy(
                table_hbm.at[idx_v.at[i, pl.ds(G1, G2)]],
                bufs.at[s, pl.ds(G1, G2)], gsem.at[s])

        def wait_gathers(s):
            pltpu.make_async_copy(
                table_hbm.at[pl.ds(0, MAX_LEN)], bufs.at[s], gsem.at[s]).wait()

        def wait_out(s):
            pltpu.make_async_copy(
                bufs.at[s], out_hbm.at[b0], osem.at[s]).wait()

        def process(i, s):
            wait_gathers(s)

            def row(r, c):
                for j in range(D_MODEL // LANES):
                    plsc.addupdate(
                        bufs.at[s, r, pl.ds(LANES * j, LANES)],
                        pe_v[r, pl.ds(LANES * j, LANES)])
                return c

            lax.fori_loop(0, MAX_LEN, row, 0)
            pltpu.async_copy(bufs.at[s], out_hbm.at[b0 + i], osem.at[s])

        def outer(io, carry):
            for s in range(NBUF):
                i = io * NBUF + s

                @pl.when(io >= 1)
                def _():
                    wait_out(s)

                start_gathers(i, s)
                if s == 0:
                    @pl.when(io >= 1)
                    def _():
                        process(io * NBUF - 1, NBUF - 1)
                else:
                    process(i - 1, s - 1)
            return carry

        lax.fori_loop(0, BPW // NBUF, outer, 0)
        process(BPW - 1, NBUF - 1)
        for s in range(NBUF):
            wait_out(s)

    return k(indices, table, pe)
